# Initial kernel scaffold; baseline (speedup 1.0000x reference)
import jax
import jax.numpy as jnp
from jax.experimental import pallas as pl


def kernel(x_t, t, cond_emb, W1, b1, W2, b2, Wc, t_table):
    raise NotImplementedError("write your pallas kernel here")


# speedup vs baseline: 8.1977x; 8.1977x over previous
import jax
import jax.numpy as jnp
from jax.experimental import pallas as pl
from jax.experimental.pallas import tpu as pltpu


_TILE = 2048


def _make_body(tiles_per_batch):
    def _mlp_body(t_ref, x_ref, tt_ref, ce_ref, Wc_ref, W1_ref, b1_ref,
                  W2_ref, b2_ref, o_ref):
        b = pl.program_id(0) // tiles_per_batch
        trow = tt_ref[pl.ds(t_ref[b], 1), :]
        cemb = jnp.dot(ce_ref[pl.ds(b, 1), :], Wc_ref[...],
                       preferred_element_type=jnp.float32)
        pre = jnp.dot(x_ref[...], W1_ref[...],
                      preferred_element_type=jnp.float32)
        h = jnp.maximum(pre + b1_ref[...] + trow + cemb, 0.0)
        o_ref[...] = jnp.dot(h, W2_ref[...],
                             preferred_element_type=jnp.float32) + b2_ref[...]
    return _mlp_body


def kernel(x_t, t, cond_emb, W1, b1, W2, b2, Wc, t_table):
    B, N, F = x_t.shape
    H = W1.shape[1]
    COND = cond_emb.shape[1]
    T = t_table.shape[0]
    x2d = x_t.reshape(B * N, F)
    tiles_per_batch = N // _TILE
    num_tiles = B * tiles_per_batch

    grid_spec = pltpu.PrefetchScalarGridSpec(
        num_scalar_prefetch=1,
        grid=(num_tiles,),
        in_specs=[
            pl.BlockSpec((_TILE, F), lambda i, t_ref: (i, 0)),
            pl.BlockSpec((T, H), lambda i, t_ref: (0, 0)),
            pl.BlockSpec((B, COND), lambda i, t_ref: (0, 0)),
            pl.BlockSpec((COND, H), lambda i, t_ref: (0, 0)),
            pl.BlockSpec((F, H), lambda i, t_ref: (0, 0)),
            pl.BlockSpec((1, H), lambda i, t_ref: (0, 0)),
            pl.BlockSpec((H, F), lambda i, t_ref: (0, 0)),
            pl.BlockSpec((1, F), lambda i, t_ref: (0, 0)),
        ],
        out_specs=pl.BlockSpec((_TILE, F), lambda i, t_ref: (i, 0)),
    )

    out2d = pl.pallas_call(
        _make_body(tiles_per_batch),
        grid_spec=grid_spec,
        out_shape=jax.ShapeDtypeStruct((B * N, F), x_t.dtype),
    )(t, x2d, t_table, cond_emb, Wc, W1, b1.reshape(1, H), W2,
      b2.reshape(1, F))
    return out2d.reshape(B, N, F)


# speedup vs baseline: 22.7811x; 2.7790x over previous
import jax
import jax.numpy as jnp
from jax.experimental import pallas as pl
from jax.experimental.pallas import tpu as pltpu


_TILE_N = 2048


def _make_body(tiles_per_batch):
    def _mlp_body(t_ref, x_ref, tt_ref, ce_ref, Wc_ref, W1_ref, b1_ref,
                  W2_ref, b2_ref, o_ref):
        b = pl.program_id(0) // tiles_per_batch
        trow = tt_ref[pl.ds(t_ref[b], 1), :]
        cemb = jnp.dot(ce_ref[pl.ds(b, 1), :], Wc_ref[...],
                       preferred_element_type=jnp.float32)
        bias_col = jnp.swapaxes(b1_ref[...] + trow + cemb, 0, 1)
        x_blk = x_ref[0]
        pre = jax.lax.dot_general(
            W1_ref[...], x_blk, (((0,), (0,)), ((), ())),
            preferred_element_type=jnp.float32)
        h = jnp.maximum(pre + bias_col, 0.0)
        out = jax.lax.dot_general(
            W2_ref[...], h, (((0,), (0,)), ((), ())),
            preferred_element_type=jnp.float32)
        o_ref[0] = out + jnp.swapaxes(b2_ref[...], 0, 1)
    return _mlp_body


def kernel(x_t, t, cond_emb, W1, b1, W2, b2, Wc, t_table):
    B, N, F = x_t.shape
    H = W1.shape[1]
    COND = cond_emb.shape[1]
    T = t_table.shape[0]
    xT = jnp.swapaxes(x_t, 1, 2)
    tiles_per_batch = N // _TILE_N
    num_tiles = B * tiles_per_batch

    grid_spec = pltpu.PrefetchScalarGridSpec(
        num_scalar_prefetch=1,
        grid=(num_tiles,),
        in_specs=[
            pl.BlockSpec((1, F, _TILE_N),
                         lambda i, t_ref: (i // tiles_per_batch, 0,
                                           i % tiles_per_batch)),
            pl.BlockSpec((T, H), lambda i, t_ref: (0, 0)),
            pl.BlockSpec((B, COND), lambda i, t_ref: (0, 0)),
            pl.BlockSpec((COND, H), lambda i, t_ref: (0, 0)),
            pl.BlockSpec((F, H), lambda i, t_ref: (0, 0)),
            pl.BlockSpec((1, H), lambda i, t_ref: (0, 0)),
            pl.BlockSpec((H, F), lambda i, t_ref: (0, 0)),
            pl.BlockSpec((1, F), lambda i, t_ref: (0, 0)),
        ],
        out_specs=pl.BlockSpec((1, F, _TILE_N),
                               lambda i, t_ref: (i // tiles_per_batch, 0,
                                                 i % tiles_per_batch)),
    )

    outT = pl.pallas_call(
        _make_body(tiles_per_batch),
        grid_spec=grid_spec,
        out_shape=jax.ShapeDtypeStruct((B, F, N), x_t.dtype),
    )(t, xT, t_table, cond_emb, Wc, W1, b1.reshape(1, H), W2,
      b2.reshape(1, F))
    return jnp.swapaxes(outT, 1, 2)
